# baseline (speedup 1.0000x reference)
import functools

import jax
import jax.numpy as jnp
from jax.experimental import pallas as pl
from jax.experimental.pallas import tpu as pltpu

K = 16
BN_EPS = 1e-5
Q = 1024
N = 100000
D = 64
TILE = 2048
N_PAD = ((N + TILE - 1) // TILE) * TILE
NUM_TILES = N_PAD // TILE


def _sim_kernel(src_ref, dst_ref, out_ref):
    j = pl.program_id(0)
    sim = jax.lax.dot_general(
        src_ref[...], dst_ref[...],
        dimension_numbers=(((1,), (1,)), ((), ())),
        preferred_element_type=jnp.float32,
    )
    col = j * TILE + jax.lax.broadcasted_iota(jnp.int32, (Q, TILE), 1)
    out_ref[...] = jnp.where(col < N, sim, -jnp.inf)


def kernel(src_emb, dst_emb, src_batch, dst_batch, bn_weight, bn_bias):
    dst_pad = jnp.pad(dst_emb, ((0, N_PAD - N), (0, 0)))
    sim = pl.pallas_call(
        _sim_kernel,
        grid=(NUM_TILES,),
        in_specs=[
            pl.BlockSpec((Q, D), lambda j: (0, 0)),
            pl.BlockSpec((TILE, D), lambda j: (j, 0)),
        ],
        out_specs=pl.BlockSpec((Q, TILE), lambda j: (0, j)),
        out_shape=jax.ShapeDtypeStruct((Q, N_PAD), jnp.float32),
    )(src_emb, dst_pad)
    top_v, top_i = jax.lax.top_k(sim, K)
    src_idx = jnp.repeat(jnp.arange(Q, dtype=top_i.dtype), K)
    graph = jnp.stack([src_idx, top_i.reshape(-1)], axis=0)
    likelihood = jnp.einsum('ij,ij->i', src_emb[graph[0]], dst_emb[graph[1]])
    mean = jnp.mean(likelihood)
    var = jnp.var(likelihood)
    logits = (likelihood - mean) / jnp.sqrt(var + BN_EPS) * bn_weight[0] + bn_bias[0]
    ew = jnp.exp(logits)
    ew = ew / jnp.mean(ew)
    return graph, ew


# speedup vs baseline: 5.7939x; 5.7939x over previous
import functools

import jax
import jax.numpy as jnp
from jax import lax
from jax.experimental import pallas as pl
from jax.experimental.pallas import tpu as pltpu

K = 16
BN_EPS = 1e-5
Q = 1024
N = 100000
D = 64
TILE = 2048
CHUNK = 256
CAP = 64
BIG = 2**30

NEG_INF = -jnp.inf
_INTERPRET = False


def _phase_a_body(dst_ref, src_ref, t_ref, m_ref):
    n_tiles = pl.num_programs(0)
    cpt = TILE // CHUNK
    j = pl.program_id(0)
    src = src_ref[...]
    rows = []
    for c in range(cpt):
        sim = lax.dot_general(
            dst_ref[pl.ds(c * CHUNK, CHUNK), :], src,
            dimension_numbers=(((1,), (1,)), ((), ())),
            preferred_element_type=jnp.float32,
        )
        gi = j * TILE + c * CHUNK + lax.broadcasted_iota(jnp.int32, (CHUNK, Q), 0)
        sim = jnp.where(gi < N, sim, NEG_INF)
        rows.append(jnp.max(sim, axis=0, keepdims=True))
    m_ref[pl.ds(j * cpt, cpt), :] = jnp.concatenate(rows, axis=0)

    @pl.when(j == n_tiles - 1)
    def _():
        def body(i, _):
            m = m_ref[...]
            mx = jnp.max(m, axis=0, keepdims=True)
            m_ref[...] = jnp.where(m == mx, NEG_INF, m)
            return 0

        lax.fori_loop(0, K - 1, body, 0)
        t = jnp.max(m_ref[...], axis=0, keepdims=True)
        t_ref[...] = jnp.broadcast_to(t, (8, Q))


def _phase_b_body(t_ref, dst_ref, src_ref, ti_ref, cv_ref, ci_ref, cnt_ref, ch_ref):
    n_tiles = pl.num_programs(0)
    cpt = TILE // CHUNK
    j = pl.program_id(0)

    @pl.when(j == 0)
    def _():
        cv_ref[...] = jnp.full((CAP, Q), NEG_INF, jnp.float32)
        ci_ref[...] = jnp.zeros((CAP, Q), jnp.int32)
        cnt_ref[...] = jnp.zeros((8, Q), jnp.int32)

    src = src_ref[...]
    t = t_ref[...][0:1, :]
    iota_cap = lax.broadcasted_iota(jnp.int32, (CAP, Q), 0)

    for c in range(cpt):
        sim = lax.dot_general(
            dst_ref[pl.ds(c * CHUNK, CHUNK), :], src,
            dimension_numbers=(((1,), (1,)), ((), ())),
            preferred_element_type=jnp.float32,
        )
        gbase = j * TILE + c * CHUNK
        gi_c = gbase + lax.broadcasted_iota(jnp.int32, (CHUNK, Q), 0)
        sim = jnp.where(gi_c < N, sim, NEG_INF)
        ch_ref[...] = sim
        m0 = jnp.max(sim, axis=0, keepdims=True)

        def cond(m):
            return jnp.any(m >= t)

        def body(m):
            ch = ch_ref[...]
            emit = m >= t
            eq = ch == m
            idx = jnp.min(jnp.where(eq, gi_c, BIG), axis=0, keepdims=True)
            cnt = cnt_ref[...]
            sel = emit & (iota_cap == cnt[0:1, :])
            cv_ref[...] = jnp.where(sel, jnp.broadcast_to(m, (CAP, Q)), cv_ref[...])
            ci_ref[...] = jnp.where(sel, jnp.broadcast_to(idx, (CAP, Q)), ci_ref[...])
            cnt_ref[...] = cnt + jnp.broadcast_to(emit.astype(jnp.int32), (8, Q))
            ch = jnp.where(eq & (gi_c == idx) & emit, NEG_INF, ch)
            ch_ref[...] = ch
            return jnp.max(ch, axis=0, keepdims=True)

        lax.while_loop(cond, body, m0)

    @pl.when(j == n_tiles - 1)
    def _():
        out_rows = []
        cv = cv_ref[...]
        ci = ci_ref[...]
        for _k in range(K):
            m = jnp.max(cv, axis=0, keepdims=True)
            eq = cv == m
            idx = jnp.min(jnp.where(eq, ci, BIG), axis=0, keepdims=True)
            out_rows.append(idx)
            cv = jnp.where(eq & (ci == idx), NEG_INF, cv)
        ti_ref[...] = jnp.concatenate(out_rows, axis=0)


def _topk_indices(src_emb, dst_pad):
    n_pad = dst_pad.shape[0]
    n_tiles = n_pad // TILE
    blocks = n_pad // CHUNK

    t8 = pl.pallas_call(
        _phase_a_body,
        grid=(n_tiles,),
        in_specs=[
            pl.BlockSpec((TILE, D), lambda j: (j, 0)),
            pl.BlockSpec((Q, D), lambda j: (0, 0)),
        ],
        out_specs=pl.BlockSpec((8, Q), lambda j: (0, 0)),
        out_shape=jax.ShapeDtypeStruct((8, Q), jnp.float32),
        scratch_shapes=[pltpu.VMEM((blocks, Q), jnp.float32)],
        interpret=_INTERPRET,
    )(dst_pad, src_emb)

    top_i = pl.pallas_call(
        _phase_b_body,
        grid=(n_tiles,),
        in_specs=[
            pl.BlockSpec((8, Q), lambda j: (0, 0)),
            pl.BlockSpec((TILE, D), lambda j: (j, 0)),
            pl.BlockSpec((Q, D), lambda j: (0, 0)),
        ],
        out_specs=pl.BlockSpec((K, Q), lambda j: (0, 0)),
        out_shape=jax.ShapeDtypeStruct((K, Q), jnp.int32),
        scratch_shapes=[
            pltpu.VMEM((CAP, Q), jnp.float32),
            pltpu.VMEM((CAP, Q), jnp.int32),
            pltpu.VMEM((8, Q), jnp.int32),
            pltpu.VMEM((CHUNK, Q), jnp.float32),
        ],
        interpret=_INTERPRET,
    )(t8, dst_pad, src_emb)
    return top_i


def kernel(src_emb, dst_emb, src_batch, dst_batch, bn_weight, bn_bias):
    n_pad = ((N + TILE - 1) // TILE) * TILE
    dst_pad = jnp.pad(dst_emb, ((0, n_pad - N), (0, 0)))
    top_i = _topk_indices(src_emb, dst_pad)
    dst_idx = top_i.T.reshape(-1)
    src_idx = jnp.repeat(jnp.arange(Q, dtype=jnp.int32), K)
    graph = jnp.stack([src_idx, dst_idx], axis=0)
    likelihood = jnp.einsum('ij,ij->i', src_emb[graph[0]], dst_emb[graph[1]])
    mean = jnp.mean(likelihood)
    var = jnp.var(likelihood)
    logits = (likelihood - mean) / jnp.sqrt(var + BN_EPS) * bn_weight[0] + bn_bias[0]
    ew = jnp.exp(logits)
    ew = ew / jnp.mean(ew)
    return graph, ew


# speedup vs baseline: 6.2681x; 1.0818x over previous
import functools

import jax
import jax.numpy as jnp
from jax import lax
from jax.experimental import pallas as pl
from jax.experimental.pallas import tpu as pltpu

K = 16
BN_EPS = 1e-5
Q = 1024
N = 100000
D = 64
TILE = 2048
CHUNK = 256
ROWS = 1024
BIG = 2**30

NEG_INF = -jnp.inf
_INTERPRET = False


def _phase_a_body(dst_ref, src_ref, t_ref, m_ref):
    n_tiles = pl.num_programs(0)
    cpt = TILE // CHUNK
    j = pl.program_id(0)
    src = src_ref[...]
    rows = []
    for c in range(cpt):
        sim = lax.dot_general(
            dst_ref[pl.ds(c * CHUNK, CHUNK), :], src,
            dimension_numbers=(((1,), (1,)), ((), ())),
            preferred_element_type=jnp.float32,
        )
        gi = j * TILE + c * CHUNK + lax.broadcasted_iota(jnp.int32, (CHUNK, Q), 0)
        sim = jnp.where(gi < N, sim, NEG_INF)
        rows.append(jnp.max(sim, axis=0, keepdims=True))
    m_ref[pl.ds(j * cpt, cpt), :] = jnp.concatenate(rows, axis=0)

    @pl.when(j == n_tiles - 1)
    def _():
        def body(i, _):
            m = m_ref[...]
            mx = jnp.max(m, axis=0, keepdims=True)
            m_ref[...] = jnp.where(m == mx, NEG_INF, m)
            return 0

        lax.fori_loop(0, K - 1, body, 0)
        t = jnp.max(m_ref[...], axis=0, keepdims=True)
        t_ref[...] = jnp.broadcast_to(t, (8, Q))


def _phase_b_body(t_ref, dst_ref, src_ref, ti_ref, rv_ref, ri_ref, s_ref, g_ref):
    n_tiles = pl.num_programs(0)
    cpt = TILE // CHUNK
    j = pl.program_id(0)

    @pl.when(j == 0)
    def _():
        rv_ref[...] = jnp.full((ROWS, Q), NEG_INF, jnp.float32)
        g_ref[0] = 0

    src = src_ref[...]
    t = t_ref[...][0:1, :]

    for c in range(cpt):
        sim = lax.dot_general(
            dst_ref[pl.ds(c * CHUNK, CHUNK), :], src,
            dimension_numbers=(((1,), (1,)), ((), ())),
            preferred_element_type=jnp.float32,
        )
        gbase = j * TILE + c * CHUNK
        gi_c = gbase + lax.broadcasted_iota(jnp.int32, (CHUNK, Q), 0)
        s = jnp.where((sim >= t) & (gi_c < N), sim, NEG_INF)
        s_ref[...] = s
        idx0 = jnp.min(jnp.where(s > NEG_INF, gi_c, BIG), axis=0, keepdims=True)

        def cond(idx):
            return jnp.any(idx < BIG)

        def body(idx):
            sc = s_ref[...]
            found = idx < BIG
            hit = gi_c == idx
            val = jnp.max(jnp.where(hit, sc, NEG_INF), axis=0, keepdims=True)
            g = jnp.minimum(g_ref[0], ROWS - 1)
            rv_ref[pl.ds(g, 1), :] = jnp.where(found, val, NEG_INF)
            ri_ref[pl.ds(g, 1), :] = idx
            g_ref[0] = g + 1
            sc = jnp.where(hit, NEG_INF, sc)
            s_ref[...] = sc
            return jnp.min(jnp.where(sc > NEG_INF, gi_c, BIG), axis=0, keepdims=True)

        lax.while_loop(cond, body, idx0)

    @pl.when(j == n_tiles - 1)
    def _():
        out_rows = []
        rv = rv_ref[...]
        ri = ri_ref[...]
        for _k in range(K):
            m = jnp.max(rv, axis=0, keepdims=True)
            eq = rv == m
            idx = jnp.min(jnp.where(eq, ri, BIG), axis=0, keepdims=True)
            out_rows.append(idx)
            rv = jnp.where(eq & (ri == idx), NEG_INF, rv)
        ti_ref[...] = jnp.concatenate(out_rows, axis=0)


def _topk_indices(src_emb, dst_pad):
    n_pad = dst_pad.shape[0]
    n_tiles = n_pad // TILE
    blocks = n_pad // CHUNK

    t8 = pl.pallas_call(
        _phase_a_body,
        grid=(n_tiles,),
        in_specs=[
            pl.BlockSpec((TILE, D), lambda j: (j, 0)),
            pl.BlockSpec((Q, D), lambda j: (0, 0)),
        ],
        out_specs=pl.BlockSpec((8, Q), lambda j: (0, 0)),
        out_shape=jax.ShapeDtypeStruct((8, Q), jnp.float32),
        scratch_shapes=[pltpu.VMEM((blocks, Q), jnp.float32)],
        interpret=_INTERPRET,
    )(dst_pad, src_emb)

    top_i = pl.pallas_call(
        _phase_b_body,
        grid=(n_tiles,),
        in_specs=[
            pl.BlockSpec((8, Q), lambda j: (0, 0)),
            pl.BlockSpec((TILE, D), lambda j: (j, 0)),
            pl.BlockSpec((Q, D), lambda j: (0, 0)),
        ],
        out_specs=pl.BlockSpec((K, Q), lambda j: (0, 0)),
        out_shape=jax.ShapeDtypeStruct((K, Q), jnp.int32),
        scratch_shapes=[
            pltpu.VMEM((ROWS, Q), jnp.float32),
            pltpu.VMEM((ROWS, Q), jnp.int32),
            pltpu.VMEM((CHUNK, Q), jnp.float32),
            pltpu.SMEM((1,), jnp.int32),
        ],
        interpret=_INTERPRET,
    )(t8, dst_pad, src_emb)
    return top_i


def kernel(src_emb, dst_emb, src_batch, dst_batch, bn_weight, bn_bias):
    n_pad = ((N + TILE - 1) // TILE) * TILE
    dst_pad = jnp.pad(dst_emb, ((0, n_pad - N), (0, 0)))
    top_i = _topk_indices(src_emb, dst_pad)
    dst_idx = top_i.T.reshape(-1)
    src_idx = jnp.repeat(jnp.arange(Q, dtype=jnp.int32), K)
    graph = jnp.stack([src_idx, dst_idx], axis=0)
    likelihood = jnp.einsum('ij,ij->i', src_emb[graph[0]], dst_emb[graph[1]])
    mean = jnp.mean(likelihood)
    var = jnp.var(likelihood)
    logits = (likelihood - mean) / jnp.sqrt(var + BN_EPS) * bn_weight[0] + bn_bias[0]
    ew = jnp.exp(logits)
    ew = ew / jnp.mean(ew)
    return graph, ew
